# Initial kernel scaffold; baseline (speedup 1.0000x reference)
import functools

import jax
import jax.numpy as jnp
from jax import lax
from jax.experimental import pallas as pl
from jax.experimental.pallas import tpu as pltpu
from jax.experimental.pallas import tpu_sc as plsc

DIM = 64
B = 16384
NEG = 20
B2 = 4096
L = 5
W = 10
G21 = NEG + 1
NC, NS = 2, 16
NW = NC * NS
CG = 32
PAIRS = CG * G21
TILE = 96
NT = PAIRS // TILE
GW = B // NW
NCW = GW // CG
GM = (B2 * W) // NW
NCM = GM // CG
B2W_ = B2 // NW
MROWS = B2W_ * L


def _bc(s, n=16):
    return lax.broadcast_in_dim(s, (n,), ())


@functools.partial(
    pl.kernel,
    out_type=(jax.ShapeDtypeStruct((B * G21,), jnp.float32),
              jax.ShapeDtypeStruct((B2 * W * G21,), jnp.float32)),
    mesh=plsc.VectorSubcoreMesh(core_axis_name="c", subcore_axis_name="s"),
    scratch_types=[
        pltpu.VMEM((PAIRS, DIM), jnp.float32),
        pltpu.VMEM((CG, DIM), jnp.float32),
        pltpu.VMEM((B2W_, DIM), jnp.float32),
        pltpu.VMEM((PAIRS,), jnp.float32),
        pltpu.VMEM((NT, TILE), jnp.int32),
        pltpu.VMEM((CG,), jnp.int32),
        pltpu.VMEM((L, 128), jnp.int32),
        pltpu.VMEM((B2W_,), jnp.int32),
        pltpu.SemaphoreType.DMA,
    ],
)
def _sc_dots(ct, xt, cw2, xw3, mw3, ml2, xm3, dw_out, dm_out,
             xvm, cvm, mvm, dvm, xidx, cidx, midx, lvm, sem):
    wid = lax.axis_index("s") * NC + lax.axis_index("c")
    iota = lax.iota(jnp.int32, 16)

    pltpu.sync_copy(mw3.at[wid], midx)
    pltpu.sync_copy(ml2.at[wid], lvm)
    cps = [pltpu.async_copy(ct.at[midx.at[t]], xvm.at[pl.ds(t * 128, 128)], sem)
           for t in range(L)]
    for cp in cps:
        cp.wait()

    def mbody(b, _):
        lnv = _bc(lvm[b])
        lnf = lnv.astype(jnp.float32)
        for dv in range(DIM // 16):
            acc = jnp.zeros((16,), jnp.float32)
            for l in range(L):
                r = xvm[b * L + l, pl.ds(dv * 16, 16)]
                m = jnp.full((16,), l, jnp.int32) < lnv
                acc = acc + jnp.where(m, r, 0.0)
            mvm[b, pl.ds(dv * 16, 16)] = acc / lnf
        return 0
    lax.fori_loop(0, B2W_, mbody, 0)

    def word_chunk(c, _):
        cglob = wid * NCW + c
        pltpu.sync_copy(cw2.at[cglob], cidx)
        pltpu.sync_copy(xw3.at[cglob], xidx)
        cc = pltpu.async_copy(ct.at[cidx], cvm, sem)
        xs = [pltpu.async_copy(xt.at[xidx.at[t]], xvm.at[pl.ds(t * TILE, TILE)], sem)
              for t in range(NT)]
        cc.wait()
        for x in xs:
            x.wait()
        for kk in range(CG // 16):
            g = kk * 16 + iota
            xbase = g * G21

            def dbody(d, accs):
                col = _bc(d)
                cd = plsc.load_gather(cvm, [g, col])
                return tuple(accs[j] + plsc.load_gather(xvm, [xbase + j, col]) * cd
                             for j in range(G21))
            accs = lax.fori_loop(
                0, DIM, dbody,
                tuple(jnp.zeros((16,), jnp.float32) for _ in range(G21)))
            for j in range(G21):
                v = -accs[0] if j == 0 else accs[j]
                plsc.store_scatter(dvm, [xbase + j], v)
        pltpu.sync_copy(dvm, dw_out.at[pl.ds(cglob * PAIRS, PAIRS)])
        return 0
    lax.fori_loop(0, NCW, word_chunk, 0)

    def mwe_chunk(c, _):
        cglob = wid * NCM + c
        pltpu.sync_copy(xm3.at[cglob], xidx)
        xs = [pltpu.async_copy(xt.at[xidx.at[t]], xvm.at[pl.ds(t * TILE, TILE)], sem)
              for t in range(NT)]
        for x in xs:
            x.wait()
        for kk in range(CG // 16):
            g = kk * 16 + iota
            gl = c * CG + g
            crow = gl // W
            xbase = g * G21

            def dbody(d, accs):
                col = _bc(d)
                cd = plsc.load_gather(mvm, [crow, col])
                return tuple(accs[j] + plsc.load_gather(xvm, [xbase + j, col]) * cd
                             for j in range(G21))
            accs = lax.fori_loop(
                0, DIM, dbody,
                tuple(jnp.zeros((16,), jnp.float32) for _ in range(G21)))
            flat0 = g * G21
            kval = plsc.load_gather(xidx, [flat0 // TILE, flat0 % TILE])
            keep = kval != 0
            neg_big = jnp.full((16,), -1e9, jnp.float32)
            for j in range(G21):
                v = -accs[0] if j == 0 else accs[j]
                v = jnp.where(keep, v, neg_big)
                plsc.store_scatter(dvm, [xbase + j], v)
        pltpu.sync_copy(dvm, dm_out.at[pl.ds(cglob * PAIRS, PAIRS)])
        return 0
    lax.fori_loop(0, NCM, mwe_chunk, 0)


def _tc_body(dw_ref, dm_ref, omw_ref, out_ref):
    def sp_sum(x):
        return jnp.sum(jnp.maximum(x, 0.0) + jnp.log(1.0 + jnp.exp(-jnp.abs(x))))
    lw = sp_sum(dw_ref[...])
    lm = sp_sum(dm_ref[...])
    cnt = jnp.sum((omw_ref[...] != 0).astype(jnp.float32))
    out_ref[0, 0] = lw / B + 25.0 * lm / cnt


def kernel(center_words, outside_words, negative_examples_words, mwe_words,
           mwe_length, outside_mwe_words, negative_examples_mwe,
           center_table, context_table):
    cw2 = center_words.reshape(B // CG, CG)
    xw3 = jnp.concatenate(
        [outside_words[:, None], negative_examples_words], axis=1
    ).reshape(B * G21 // PAIRS, NT, TILE)
    mw3 = mwe_words.reshape(NW, L, 128)
    ml2 = mwe_length.reshape(NW, B2W_)
    xm3 = jnp.concatenate(
        [outside_mwe_words.reshape(-1, 1), negative_examples_mwe], axis=1
    ).reshape(B2 * W * G21 // PAIRS, NT, TILE)

    dw, dm = _sc_dots(center_table, context_table, cw2, xw3, mw3, ml2, xm3)

    out = pl.pallas_call(
        _tc_body,
        out_shape=jax.ShapeDtypeStruct((1, 1), jnp.float32),
    )(dw.reshape(B * G21 // 128, 128),
      dm.reshape(B2 * W * G21 // 128, 128),
      outside_mwe_words.reshape(B2 * W // 128, 128))
    return out[0, 0]


# speedup vs baseline: 7.5495x; 7.5495x over previous
import functools

import jax
import jax.numpy as jnp
from jax import lax
from jax.experimental import pallas as pl
from jax.experimental.pallas import tpu as pltpu
from jax.experimental.pallas import tpu_sc as plsc

DIM = 64
B = 16384
NEG = 20
B2 = 4096
L = 5
W = 10
G21 = NEG + 1
NC, NS = 2, 16
NW = NC * NS
CG = 32
PAIRS = CG * G21
TILE = 96
NT = PAIRS // TILE
GW = B // NW
NCW = GW // CG
GM = (B2 * W) // NW
NCM = GM // CG
B2W_ = B2 // NW
MROWS = B2W_ * L


def _bc(s, n=16):
    return lax.broadcast_in_dim(s, (n,), ())


@functools.partial(
    pl.kernel,
    out_type=(jax.ShapeDtypeStruct((B * G21,), jnp.float32),
              jax.ShapeDtypeStruct((B2 * W * G21,), jnp.float32)),
    mesh=plsc.VectorSubcoreMesh(core_axis_name="c", subcore_axis_name="s"),
    compiler_params=pltpu.CompilerParams(
        use_tc_tiling_on_sc=False, needs_layout_passes=False),
    scratch_types=[
        pltpu.VMEM((PAIRS, DIM), jnp.float32),
        pltpu.VMEM((CG, DIM), jnp.float32),
        pltpu.VMEM((B2W_, DIM), jnp.float32),
        pltpu.VMEM((PAIRS,), jnp.float32),
        pltpu.VMEM((NT, TILE), jnp.int32),
        pltpu.VMEM((CG,), jnp.int32),
        pltpu.VMEM((L, 128), jnp.int32),
        pltpu.VMEM((B2W_,), jnp.int32),
        pltpu.SemaphoreType.DMA,
    ],
)
def _sc_dots(ct, xt, cw2, xw3, mw3, ml2, xm3, dw_out, dm_out,
             xvm, cvm, mvm, dvm, xidx, cidx, midx, lvm, sem):
    wid = lax.axis_index("s") * NC + lax.axis_index("c")
    iota = lax.iota(jnp.int32, 16)

    pltpu.sync_copy(mw3.at[wid], midx)
    pltpu.sync_copy(ml2.at[wid], lvm)
    cps = [pltpu.async_copy(ct.at[midx.at[t]], xvm.at[pl.ds(t * 128, 128)], sem)
           for t in range(L)]
    for cp in cps:
        cp.wait()

    for bb in range(B2W_ // 16):
        b = bb * 16 + iota
        lnv = lvm[pl.ds(bb * 16, 16)]
        lnf = lnv.astype(jnp.float32)

        def mbody(d, _):
            col = _bc(d)
            acc = jnp.zeros((16,), jnp.float32)
            for l in range(L):
                r = plsc.load_gather(xvm, [b * L + l, col])
                m = jnp.full((16,), l, jnp.int32) < lnv
                acc = acc + jnp.where(m, r, 0.0)
            plsc.store_scatter(mvm, [b, col], acc / lnf)
            return 0
        lax.fori_loop(0, DIM, mbody, 0)

    def word_chunk(c, _):
        cglob = wid * NCW + c
        pltpu.sync_copy(cw2.at[cglob], cidx)
        pltpu.sync_copy(xw3.at[cglob], xidx)
        cc = pltpu.async_copy(ct.at[cidx], cvm, sem)
        xs = [pltpu.async_copy(xt.at[xidx.at[t]], xvm.at[pl.ds(t * TILE, TILE)], sem)
              for t in range(NT)]
        cc.wait()
        for x in xs:
            x.wait()
        for kk in range(CG // 16):
            g = kk * 16 + iota
            xbase = g * G21

            def dbody(d, accs):
                col = _bc(d)
                cd = plsc.load_gather(cvm, [g, col])
                return tuple(accs[j] + plsc.load_gather(xvm, [xbase + j, col]) * cd
                             for j in range(G21))
            accs = lax.fori_loop(
                0, DIM, dbody,
                tuple(jnp.zeros((16,), jnp.float32) for _ in range(G21)))
            for j in range(G21):
                v = -accs[0] if j == 0 else accs[j]
                plsc.store_scatter(dvm, [xbase + j], v)
        pltpu.sync_copy(dvm, dw_out.at[pl.ds(cglob * PAIRS, PAIRS)])
        return 0
    lax.fori_loop(0, NCW, word_chunk, 0)

    def mwe_chunk(c, _):
        cglob = wid * NCM + c
        pltpu.sync_copy(xm3.at[cglob], xidx)
        xs = [pltpu.async_copy(xt.at[xidx.at[t]], xvm.at[pl.ds(t * TILE, TILE)], sem)
              for t in range(NT)]
        for x in xs:
            x.wait()
        for kk in range(CG // 16):
            g = kk * 16 + iota
            gl = c * CG + g
            crow = gl // W
            xbase = g * G21

            def dbody(d, accs):
                col = _bc(d)
                cd = plsc.load_gather(mvm, [crow, col])
                return tuple(accs[j] + plsc.load_gather(xvm, [xbase + j, col]) * cd
                             for j in range(G21))
            accs = lax.fori_loop(
                0, DIM, dbody,
                tuple(jnp.zeros((16,), jnp.float32) for _ in range(G21)))
            flat0 = g * G21
            kval = plsc.load_gather(xidx, [flat0 // TILE, flat0 % TILE])
            keep = kval != 0
            neg_big = jnp.full((16,), -1e9, jnp.float32)
            for j in range(G21):
                v = -accs[0] if j == 0 else accs[j]
                v = jnp.where(keep, v, neg_big)
                plsc.store_scatter(dvm, [xbase + j], v)
        pltpu.sync_copy(dvm, dm_out.at[pl.ds(cglob * PAIRS, PAIRS)])
        return 0
    lax.fori_loop(0, NCM, mwe_chunk, 0)


def _tc_body(dw_ref, dm_ref, omw_ref, out_ref):
    def sp_sum(x):
        return jnp.sum(jnp.maximum(x, 0.0) + jnp.log(1.0 + jnp.exp(-jnp.abs(x))))
    lw = sp_sum(dw_ref[...])
    lm = sp_sum(dm_ref[...])
    cnt = jnp.sum((omw_ref[...] != 0).astype(jnp.float32))
    out_ref[...] = jnp.reshape(lw / B + 25.0 * lm / cnt, (1, 1))


def kernel(center_words, outside_words, negative_examples_words, mwe_words,
           mwe_length, outside_mwe_words, negative_examples_mwe,
           center_table, context_table):
    cw2 = center_words.reshape(B // CG, CG)
    xw3 = jnp.concatenate(
        [outside_words[:, None], negative_examples_words], axis=1
    ).reshape(B * G21 // PAIRS, NT, TILE)
    mw3 = mwe_words.reshape(NW, L, 128)
    ml2 = mwe_length.reshape(NW, B2W_)
    xm3 = jnp.concatenate(
        [outside_mwe_words.reshape(-1, 1), negative_examples_mwe], axis=1
    ).reshape(B2 * W * G21 // PAIRS, NT, TILE)

    dw, dm = _sc_dots(center_table, context_table, cw2, xw3, mw3, ml2, xm3)

    out = pl.pallas_call(
        _tc_body,
        out_shape=jax.ShapeDtypeStruct((1, 1), jnp.float32),
    )(dw.reshape(B * G21 // 128, 128),
      dm.reshape(B2 * W * G21 // 128, 128),
      outside_mwe_words.reshape(B2 * W // 128, 128))
    return out[0, 0]


# speedup vs baseline: 7.9030x; 1.0468x over previous
import functools

import jax
import jax.numpy as jnp
from jax import lax
from jax.experimental import pallas as pl
from jax.experimental.pallas import tpu as pltpu
from jax.experimental.pallas import tpu_sc as plsc

DIM = 64
B = 16384
NEG = 20
B2 = 4096
L = 5
W = 10
NC, NS = 2, 16
NW = NC * NS
CG = 32
NROWS = CG * NEG
NTN = NROWS // 128
NCW = B // NW // CG
NCM = (B2 * W) // NW // CG
B2W_ = B2 // NW


def _bc(s, n=16):
    return lax.broadcast_in_dim(s, (n,), ())


@functools.partial(
    pl.kernel,
    out_type=(jax.ShapeDtypeStruct((B * NEG,), jnp.float32),
              jax.ShapeDtypeStruct((B,), jnp.float32),
              jax.ShapeDtypeStruct((B2 * W * NEG,), jnp.float32),
              jax.ShapeDtypeStruct((B2 * W,), jnp.float32)),
    mesh=plsc.VectorSubcoreMesh(core_axis_name="c", subcore_axis_name="s"),
    compiler_params=pltpu.CompilerParams(
        use_tc_tiling_on_sc=False, needs_layout_passes=False),
    scratch_types=[
        pltpu.VMEM((2, NROWS, DIM), jnp.float32),
        pltpu.VMEM((2, CG, DIM), jnp.float32),
        pltpu.VMEM((2, CG, DIM), jnp.float32),
        pltpu.VMEM((B2W_, DIM), jnp.float32),
        pltpu.VMEM((2, NTN, 128), jnp.int32),
        pltpu.VMEM((2, CG), jnp.int32),
        pltpu.VMEM((2, CG), jnp.int32),
        pltpu.VMEM((2, NROWS), jnp.float32),
        pltpu.VMEM((NCM * CG,), jnp.float32),
        pltpu.VMEM((B2W_,), jnp.int32),
        pltpu.SemaphoreType.DMA,
        pltpu.SemaphoreType.DMA,
        pltpu.SemaphoreType.DMA,
    ],
)
def _sc_dots(ct, xt, cw2, ow2, nw2, mw3, ml2, om2, nm2,
             dnw_out, dpw_out, dnm_out, dpm_out,
             nvm, pvm, cvm, mvm, nidx, pidx, cidx, dnvm, dp_all, lvm,
             sem_i, sem_g, sem_w):
    wid = lax.axis_index("s") * NC + lax.axis_index("c")
    iota = lax.iota(jnp.int32, 16)

    pltpu.sync_copy(mw3.at[wid], nidx.at[0])
    pltpu.sync_copy(ml2.at[wid], lvm)
    cps = [pltpu.async_copy(ct.at[nidx.at[0, t]],
                            nvm.at[0, pl.ds(t * 128, 128)], sem_g)
           for t in range(L)]
    for cp in cps:
        cp.wait()
    for bb in range(B2W_ // 16):
        bv = bb * 16 + iota
        lnv = lvm[pl.ds(bb * 16, 16)]
        lnf = lnv.astype(jnp.float32)

        def mbody(d, _):
            col = _bc(d)
            acc = jnp.zeros((16,), jnp.float32)
            for l in range(L):
                r = plsc.load_gather(nvm, [_bc(0), bv * L + l, col])
                m = jnp.full((16,), l, jnp.int32) < lnv
                acc = acc + jnp.where(m, r, 0.0)
            plsc.store_scatter(mvm, [bv, col], acc / lnf)
            return 0
        lax.fori_loop(0, DIM, mbody, 0)

    def run_phase(ncc, is_word):
        cbase = wid * ncc

        def idx_copies(c, buf):
            cglob = cbase + c
            ops = [pltpu.make_async_copy(
                nw2.at[pl.ds(cglob * NTN, NTN)] if is_word
                else nm2.at[pl.ds(cglob * NTN, NTN)],
                nidx.at[buf], sem_i)]
            prow, pcol = cglob // 4, (cglob % 4) * CG
            ops.append(pltpu.make_async_copy(
                (ow2 if is_word else om2).at[prow, pl.ds(pcol, CG)],
                pidx.at[buf], sem_i))
            if is_word:
                ops.append(pltpu.make_async_copy(
                    cw2.at[prow, pl.ds(pcol, CG)], cidx.at[buf], sem_i))
            return ops

        def row_gathers(c, buf):
            ops = [pltpu.make_async_copy(
                xt.at[nidx.at[buf, t]],
                nvm.at[buf, pl.ds(t * 128, 128)], sem_g)
                for t in range(NTN)]
            ops.append(pltpu.make_async_copy(
                xt.at[pidx.at[buf]], pvm.at[buf], sem_g))
            if is_word:
                ops.append(pltpu.make_async_copy(
                    ct.at[cidx.at[buf]], cvm.at[buf], sem_g))
            return ops

        def dn_writeback(c, buf):
            cglob = cbase + c
            return pltpu.make_async_copy(
                dnvm.at[buf],
                (dnw_out if is_word else dnm_out).at[pl.ds(cglob * NROWS, NROWS)],
                sem_w)

        for op in idx_copies(0, 0):
            op.start()
            op.wait()
        if ncc > 1:
            for op in idx_copies(1, 1):
                op.start()
                op.wait()
        for op in row_gathers(0, 0):
            op.start()

        def chunk_body(c, _):
            buf = lax.rem(c, 2)
            nbuf = lax.rem(c + 1, 2)

            @pl.when((c >= 1) & (c + 1 < ncc))
            def _():
                for op in idx_copies(c + 1, nbuf):
                    op.wait()

            @pl.when(c + 1 < ncc)
            def _():
                for op in row_gathers(c + 1, nbuf):
                    op.start()

            for op in row_gathers(c, buf):
                op.wait()

            @pl.when(c + 2 < ncc)
            def _():
                for op in idx_copies(c + 2, buf):
                    op.start()

            @pl.when(c >= 2)
            def _():
                dn_writeback(c - 2, buf).wait()

            for kk in range(CG // 16):
                g = kk * 16 + iota
                if is_word:
                    crow = g
                else:
                    crow = (c * CG + g) // W
                nbase = g * NEG
                bufv = _bc(buf)

                def dbody(d, accs):
                    col = _bc(d)
                    if is_word:
                        cd = plsc.load_gather(cvm, [bufv, crow, col])
                    else:
                        cd = plsc.load_gather(mvm, [crow, col])
                    new = tuple(
                        accs[j] + plsc.load_gather(nvm, [bufv, nbase + j, col]) * cd
                        for j in range(NEG))
                    pd = accs[NEG] + plsc.load_gather(pvm, [bufv, g, col]) * cd
                    return new + (pd,)
                accs = lax.fori_loop(
                    0, DIM, dbody,
                    tuple(jnp.zeros((16,), jnp.float32) for _ in range(NEG + 1)))
                if is_word:
                    for j in range(NEG):
                        plsc.store_scatter(dnvm, [bufv, nbase + j], accs[j])
                    plsc.store_scatter(dp_all, [c * CG + g], -accs[NEG])
                else:
                    kval = plsc.load_gather(pidx, [bufv, g])
                    keep = kval != 0
                    neg_big = jnp.full((16,), -1e9, jnp.float32)
                    for j in range(NEG):
                        v = jnp.where(keep, accs[j], neg_big)
                        plsc.store_scatter(dnvm, [bufv, nbase + j], v)
                    vp = jnp.where(keep, -accs[NEG], neg_big)
                    plsc.store_scatter(dp_all, [c * CG + g], vp)
            dn_writeback(c, buf).start()
            return 0
        lax.fori_loop(0, ncc, chunk_body, 0)

        if ncc >= 2:
            dn_writeback(ncc - 2, (ncc - 2) % 2).wait()
        dn_writeback(ncc - 1, (ncc - 1) % 2).wait()
        pw_out = dpw_out if is_word else dpm_out
        pltpu.sync_copy(dp_all.at[pl.ds(0, ncc * CG)],
                        pw_out.at[pl.ds(cbase * CG, ncc * CG)])

    run_phase(NCW, True)
    run_phase(NCM, False)


def _tc_body(dnw_ref, dpw_ref, dnm_ref, dpm_ref, omw_ref, out_ref):
    def sp_sum(x):
        return jnp.sum(jnp.maximum(x, 0.0) + jnp.log(1.0 + jnp.exp(-jnp.abs(x))))
    lw = sp_sum(dnw_ref[...]) + sp_sum(dpw_ref[...])
    lm = sp_sum(dnm_ref[...]) + sp_sum(dpm_ref[...])
    cnt = jnp.sum((omw_ref[...] != 0).astype(jnp.float32))
    out_ref[...] = jnp.reshape(lw / B + 25.0 * lm / cnt, (1, 1))


def kernel(center_words, outside_words, negative_examples_words, mwe_words,
           mwe_length, outside_mwe_words, negative_examples_mwe,
           center_table, context_table):
    cw2 = center_words.reshape(B // 128, 128)
    ow2 = outside_words.reshape(B // 128, 128)
    nw2 = negative_examples_words.reshape(B * NEG // 128, 128)
    mw3 = mwe_words.reshape(NW, L, 128)
    ml2 = mwe_length.reshape(NW, B2W_)
    om2 = outside_mwe_words.reshape(B2 * W // 128, 128)
    nm2 = negative_examples_mwe.reshape(B2 * W * NEG // 128, 128)

    dnw, dpw, dnm, dpm = _sc_dots(center_table, context_table,
                                  cw2, ow2, nw2, mw3, ml2, om2, nm2)

    out = pl.pallas_call(
        _tc_body,
        out_shape=jax.ShapeDtypeStruct((1, 1), jnp.float32),
    )(dnw.reshape(B * NEG // 128, 128),
      dpw.reshape(B // 128, 128),
      dnm.reshape(B2 * W * NEG // 128, 128),
      dpm.reshape(B2 * W // 128, 128),
      outside_mwe_words.reshape(B2 * W // 128, 128))
    return out[0, 0]
